# baseline (speedup 1.0000x reference)
import functools

import jax
import jax.numpy as jnp
from jax import lax
from jax.experimental import pallas as pl
from jax.experimental.pallas import tpu as pltpu
from jax.experimental.pallas import tpu_sc as plsc

MASKED_ID = 103
LN_EPS = 1e-12
S_BLK = 512

_NC = 2
_NS = 16
_NW = _NC * _NS


def _make_sc_probe(B, S, H):
    mesh = plsc.VectorSubcoreMesh(core_axis_name="c", subcore_axis_name="s")
    s_per_w = S // _NW
    n_grp = s_per_w // 8

    @functools.partial(
        pl.kernel, mesh=mesh,
        out_type=jax.ShapeDtypeStruct((_NW, 16), jnp.float32),
        compiler_params=pltpu.CompilerParams(needs_layout_passes=False),
        scratch_types=[
            pltpu.VMEM((8, H), jnp.float32),
            pltpu.VMEM((8, H), jnp.float32),
            pltpu.VMEM((16,), jnp.float32),
            pltpu.SemaphoreType.DMA,
            pltpu.SemaphoreType.DMA,
        ],
    )
    def _probe(emb_hbm, out_hbm, buf0, buf1, accv, sem0, sem1):
        wid = lax.axis_index("s") * _NC + lax.axis_index("c")
        base = wid * s_per_w
        accv[...] = jnp.zeros((16,), jnp.float32)
        bufs = (buf0, buf1)
        sems = (sem0, sem1)

        def src(g):
            b = 2 + g // n_grp
            return emb_hbm.at[b, pl.ds(base + (g % n_grp) * 8, 8), :]

        n_tot = 2 * n_grp
        copies = [None] * n_tot
        copies[0] = pltpu.async_copy(src(0), bufs[0], sems[0])
        for g in range(n_tot):
            if g + 1 < n_tot:
                copies[g + 1] = pltpu.async_copy(
                    src(g + 1), bufs[(g + 1) % 2], sems[(g + 1) % 2])
            copies[g].wait()
            buf = bufs[g % 2]
            for r in range(8):
                @pl.loop(0, H // 16)
                def _(c):
                    accv[...] = accv[...] + buf[r, pl.ds(c * 16, 16)]
        pltpu.sync_copy(accv, out_hbm.at[wid])

    return _probe


def _body(det_ref, emb_ref, pos_ref, word_ref, type_ref, gam_ref, bet_ref,
          hid_ref, sco_ref):
    row = word_ref[MASKED_ID % 8:MASKED_ID % 8 + 1, :] + type_ref[0:1, :]
    x = pos_ref[...] + row
    mean = jnp.mean(x, axis=1, keepdims=True)
    d = x - mean
    var = jnp.mean(d * d, axis=1, keepdims=True)
    m = d * jax.lax.rsqrt(var + LN_EPS) * gam_ref[...] + bet_ref[...]
    sl = det_ref[...][:, 0, :]
    sco_ref[:, 0:1, :] = (1.0 - sl)[:, None, :]
    sco_ref[:, 1:2, :] = sl[:, None, :]
    s = sl[:, :, None]
    hid_ref[...] = (1.0 - s) * emb_ref[...] + s * m[None]


def kernel(detector_scores, embeddings, word_table, pos_table, type_table,
           ln_gamma, ln_beta):
    B, S, _ = detector_scores.shape
    H = embeddings.shape[-1]
    n = S // S_BLK
    gamma2 = ln_gamma.reshape(1, H)
    beta2 = ln_beta.reshape(1, H)
    det2 = detector_scores.transpose(0, 2, 1)
    wblk = MASKED_ID // 8

    grid_spec = pl.GridSpec(
        grid=(n,),
        in_specs=[
            pl.BlockSpec((B, 1, S_BLK), lambda i: (0, 0, i)),
            pl.BlockSpec((B, S_BLK, H), lambda i: (0, i, 0)),
            pl.BlockSpec((S_BLK, H), lambda i: (i, 0)),
            pl.BlockSpec((8, H), lambda i: (wblk, 0)),
            pl.BlockSpec((2, H), lambda i: (0, 0)),
            pl.BlockSpec((1, H), lambda i: (0, 0)),
            pl.BlockSpec((1, H), lambda i: (0, 0)),
        ],
        out_specs=[
            pl.BlockSpec((B, S_BLK, H), lambda i: (0, i, 0)),
            pl.BlockSpec((B, 2, S_BLK), lambda i: (0, 0, i)),
        ],
    )
    hidden, scores_t = pl.pallas_call(
        _body,
        grid_spec=grid_spec,
        out_shape=[
            jax.ShapeDtypeStruct((B, S, H), jnp.float32),
            jax.ShapeDtypeStruct((B, 2, S), jnp.float32),
        ],
    )(det2, embeddings, pos_table, word_table, type_table, gamma2, beta2)
    probe = _make_sc_probe(B, S, H)(embeddings)
    scores_t = scores_t + 0.0 * probe[0, 0]
    return (hidden, scores_t.transpose(0, 2, 1))


# speedup vs baseline: 2.9143x; 2.9143x over previous
import jax
import jax.numpy as jnp
from jax.experimental import pallas as pl

MASKED_ID = 103
LN_EPS = 1e-12
S_BLK = 512


def _body(det_ref, emb_ref, pos_ref, word_ref, type_ref, gam_ref, bet_ref,
          hid_ref, sco_ref):
    row = word_ref[MASKED_ID % 8:MASKED_ID % 8 + 1, :] + type_ref[0:1, :]
    x = pos_ref[...] + row
    mean = jnp.mean(x, axis=1, keepdims=True)
    d = x - mean
    var = jnp.mean(d * d, axis=1, keepdims=True)
    m = d * jax.lax.rsqrt(var + LN_EPS) * gam_ref[...] + bet_ref[...]
    sl = det_ref[...][:, 0, :]
    sco_ref[:, 0:1, :] = (1.0 - sl)[:, None, :]
    sco_ref[:, 1:2, :] = sl[:, None, :]
    s = sl[:, :, None]
    hid_ref[...] = (1.0 - s) * emb_ref[...] + s * m[None]


def kernel(detector_scores, embeddings, word_table, pos_table, type_table,
           ln_gamma, ln_beta):
    B, S, _ = detector_scores.shape
    H = embeddings.shape[-1]
    n = S // S_BLK
    gamma2 = ln_gamma.reshape(1, H)
    beta2 = ln_beta.reshape(1, H)
    det2 = detector_scores.transpose(0, 2, 1)
    wblk = MASKED_ID // 8

    grid_spec = pl.GridSpec(
        grid=(n,),
        in_specs=[
            pl.BlockSpec((B, 1, S_BLK), lambda i: (0, 0, i)),
            pl.BlockSpec((B, S_BLK, H), lambda i: (0, i, 0)),
            pl.BlockSpec((S_BLK, H), lambda i: (i, 0)),
            pl.BlockSpec((8, H), lambda i: (wblk, 0)),
            pl.BlockSpec((2, H), lambda i: (0, 0)),
            pl.BlockSpec((1, H), lambda i: (0, 0)),
            pl.BlockSpec((1, H), lambda i: (0, 0)),
        ],
        out_specs=[
            pl.BlockSpec((B, S_BLK, H), lambda i: (0, i, 0)),
            pl.BlockSpec((B, 2, S_BLK), lambda i: (0, 0, i)),
        ],
    )
    hidden, scores_t = pl.pallas_call(
        _body,
        grid_spec=grid_spec,
        out_shape=[
            jax.ShapeDtypeStruct((B, S, H), jnp.float32),
            jax.ShapeDtypeStruct((B, 2, S), jnp.float32),
        ],
    )(det2, embeddings, pos_table, word_table, type_table, gamma2, beta2)
    return (hidden, scores_t.transpose(0, 2, 1))
